# baseline (speedup 1.0000x reference)
import functools
import jax
import jax.numpy as jnp
from jax import lax
from jax.experimental import pallas as pl
from jax.experimental.pallas import tpu as pltpu
from jax.experimental.pallas import tpu_sc as plsc

_F = 40
_NCHUNK = 1000 // _F


def _sc_mel_body(melt_h, melp_h, post_h, mmask_h, out_h,
                 bt, bp, bo, mbuf, obuf, sems):
    B, T_mel, n_mels = melt_h.shape
    nv = n_mels // 16
    wid = lax.axis_index("s") * 2 + lax.axis_index("c")
    b = wid

    pltpu.sync_copy(mmask_h.at[pl.ds(b * T_mel, T_mel)],
                    mbuf.at[pl.ds(0, T_mel)])

    def start(c, slot):
        f0 = c * _F
        return [
            pltpu.async_copy(melt_h.at[b, pl.ds(f0, _F), :], bt.at[slot],
                             sems.at[slot, 0]),
            pltpu.async_copy(melp_h.at[b, pl.ds(f0, _F), :], bp.at[slot],
                             sems.at[slot, 1]),
            pltpu.async_copy(post_h.at[b, pl.ds(f0, _F), :], bo.at[slot],
                             sems.at[slot, 2]),
        ]

    pending = {0: start(0, 0)}

    zero = jnp.zeros((16,), jnp.float32)
    accs = (zero, zero, zero)

    for c in range(_NCHUNK):
        slot = c % 2
        if c + 1 < _NCHUNK:
            pending[c + 1] = start(c + 1, 1 - slot)
        for h in pending.pop(c):
            h.wait()

        def frame_body(i, carry, slot=slot, c=c):
            accp, accq, accm = carry
            mv16 = mbuf[pl.ds(c * _F + i, 16)]
            mv = jnp.full((16,), mv16[0], jnp.float32)
            sp = None
            sq = None
            for k in range(nv):
                tv = bt[slot, i, pl.ds(k * 16, 16)]
                dp = jnp.abs(bp[slot, i, pl.ds(k * 16, 16)] - tv)
                dq = jnp.abs(bo[slot, i, pl.ds(k * 16, 16)] - tv)
                sp = dp if sp is None else sp + dp
                sq = dq if sq is None else sq + dq
            return (accp + mv * sp, accq + mv * sq, accm + mv)

        accs = lax.fori_loop(0, _F, frame_body, accs)

    obuf[pl.ds(0, 16)] = accs[0]
    obuf[pl.ds(16, 16)] = accs[1]
    obuf[pl.ds(32, 16)] = accs[2]
    pltpu.sync_copy(obuf, out_h.at[wid])


def _sc_mel_sums(melt, melp, post, mmask_flat):
    B, T_mel, n_mels = melt.shape
    mesh = plsc.VectorSubcoreMesh(core_axis_name="c", subcore_axis_name="s")
    kfn = pl.kernel(
        _sc_mel_body,
        out_type=jax.ShapeDtypeStruct((B, 128), jnp.float32),
        mesh=mesh,
        scratch_types=[
            pltpu.VMEM((2, _F, n_mels), jnp.float32),
            pltpu.VMEM((2, _F, n_mels), jnp.float32),
            pltpu.VMEM((2, _F, n_mels), jnp.float32),
            pltpu.VMEM((T_mel + 24,), jnp.float32),
            pltpu.VMEM((128,), jnp.float32),
            pltpu.SemaphoreType.DMA((2, 3)),
        ],
        compiler_params=pltpu.CompilerParams(use_tc_tiling_on_sc=True),
    )
    return kfn(melt, melp, post, mmask_flat)


def _tc_text_body(pt_ref, pp_ref, et_ref, ep_ref, ldp_ref, dur_ref, tm_ref,
                  out_ref):
    tm = tm_ref[...]
    pe = (pp_ref[...] - pt_ref[...]) ** 2
    ee = (ep_ref[...] - et_ref[...]) ** 2
    ldt = jnp.log(dur_ref[...] + 1.0)
    de = (ldp_ref[...] - ldt) ** 2
    out_ref[0] = jnp.sum(pe * tm)
    out_ref[1] = jnp.sum(ee * tm)
    out_ref[2] = jnp.sum(de * tm)
    out_ref[3] = jnp.sum(tm)


def kernel(mel_targets, pitch_targets, energy_targets, duration_targets,
           mel_predictions, postnet_mel_predictions, pitch_predictions,
           energy_predictions, log_duration_predictions, text_masks, mel_masks):
    B, T_mel, n_mels = mel_targets.shape

    tm = jnp.logical_not(text_masks).astype(jnp.float32)
    mm_flat = jnp.logical_not(mel_masks).astype(jnp.float32).reshape(B * T_mel)
    dur_f = duration_targets.astype(jnp.float32)

    parts = _sc_mel_sums(mel_targets, mel_predictions,
                         postnet_mel_predictions, mm_flat)

    tsums = pl.pallas_call(
        _tc_text_body,
        out_specs=pl.BlockSpec(memory_space=pltpu.SMEM),
        out_shape=jax.ShapeDtypeStruct((4,), jnp.float32),
    )(pitch_targets, pitch_predictions, energy_targets, energy_predictions,
      log_duration_predictions, dur_f, tm)

    mel_num = jnp.sum(parts[:, 0:16])
    post_num = jnp.sum(parts[:, 16:32])
    mel_msum = jnp.sum(parts[:, 32:48]) / 16.0

    n_mels_f = jnp.float32(n_mels)
    mel_loss = mel_num / (mel_msum * n_mels_f)
    postnet_mel_loss = post_num / (mel_msum * n_mels_f)
    pitch_loss = tsums[0] / tsums[3]
    energy_loss = tsums[1] / tsums[3]
    duration_loss = tsums[2] / tsums[3]
    total_loss = (mel_loss + postnet_mel_loss + duration_loss
                  + pitch_loss + energy_loss)
    return (total_loss, mel_loss, postnet_mel_loss, pitch_loss,
            energy_loss, duration_loss)
